# baseline (speedup 1.0000x reference)
import functools

import jax
import jax.numpy as jnp
from jax import lax
from jax.experimental import pallas as pl
from jax.experimental.pallas import tpu as pltpu
from jax.experimental.pallas import tpu_sc as plsc

NUM_BITS = 8
EMB_DIM = 128
NUM_CODES = 1 << NUM_BITS


def _prep_body(tables_ref, bits_ref, bv_ref, wpow_ref, lut_ref, codes_ref):
    t0 = tables_ref[0]
    t1 = tables_ref[1]
    diff = t1 - t0
    base = jnp.sum(t0, axis=0, keepdims=True)
    lut_ref[...] = (
        jnp.dot(bits_ref[...], diff, preferred_element_type=jnp.float32,
                precision=lax.Precision.HIGHEST) + base
    )
    bv = bv_ref[...].astype(jnp.float32)
    codes_f = jnp.dot(bv, wpow_ref[...], preferred_element_type=jnp.float32)
    codes_ref[...] = codes_f.astype(jnp.int32)


def _prep(tables_t, bits256, bv_rows, wpow):
    return pl.pallas_call(
        _prep_body,
        out_shape=(
            jax.ShapeDtypeStruct((NUM_CODES, EMB_DIM), jnp.float32),
            jax.ShapeDtypeStruct(bv_rows.shape[:1] * 2, jnp.int32),
        ),
    )(tables_t, bits256, bv_rows, wpow)


def _make_sc_kernel(batch):
    info = plsc.get_sparse_core_info()
    nc, ns = info.num_cores, info.num_subcores
    nw = nc * ns
    b_per_w = batch // nw
    idx_chunk = 128
    n_chunks = b_per_w // idx_chunk

    mesh = plsc.VectorSubcoreMesh(core_axis_name="c", subcore_axis_name="s")

    @functools.partial(
        pl.kernel,
        mesh=mesh,
        out_type=jax.ShapeDtypeStruct((batch, EMB_DIM), jnp.float32),
        scratch_types=[
            pltpu.VMEM((b_per_w,), jnp.int32),
            pltpu.VMEM((b_per_w, EMB_DIM), jnp.float32),
            pltpu.SemaphoreType.DMA,
            pltpu.SemaphoreType.DMA,
        ],
    )
    def sc_kernel(codes_hbm, lut_hbm, out_hbm, code_v, rows_v, sem_g, sem_w):
        wid = lax.axis_index("s") * nc + lax.axis_index("c")
        base = wid * b_per_w
        pltpu.sync_copy(codes_hbm.at[pl.ds(base, b_per_w)], code_v)
        gathers = [
            pltpu.async_copy(
                lut_hbm.at[code_v.at[pl.ds(j * idx_chunk, idx_chunk)]],
                rows_v.at[pl.ds(j * idx_chunk, idx_chunk)],
                sem_g,
            )
            for j in range(n_chunks)
        ]
        for gth in gathers:
            gth.wait()
        pltpu.async_copy(rows_v, out_hbm.at[pl.ds(base, b_per_w)],
                         sem_w).wait()

    return sc_kernel


@jax.jit
def kernel(bitvecs, tables):
    batch = bitvecs.shape[0]

    n_rows = 128
    per_row = batch // n_rows
    bv_rows = bitvecs.astype(jnp.int32).reshape(n_rows, per_row * NUM_BITS)
    tables_t = tables.astype(jnp.float32).transpose(1, 0, 2)

    codes = lax.iota(jnp.int32, NUM_CODES)[:, None]
    shifts = lax.iota(jnp.int32, NUM_BITS)[None, :]
    bits256 = ((codes >> shifts) & 1).astype(jnp.float32)
    r = lax.iota(jnp.int32, per_row * NUM_BITS)[:, None]
    c = lax.iota(jnp.int32, per_row)[None, :]
    wpow = jnp.where(r // NUM_BITS == c,
                     (1 << (r % NUM_BITS)).astype(jnp.float32), 0.0)

    lut, codes2d = _prep(tables_t, bits256, bv_rows, wpow)
    out = _make_sc_kernel(batch)(codes2d.reshape(batch), lut)
    return out
